# baseline (speedup 1.0000x reference)
import functools

import jax
import jax.numpy as jnp
from jax import lax
from jax.experimental import pallas as pl
from jax.experimental.pallas import tpu as pltpu
from jax.experimental.pallas import tpu_sc as plsc

NCORES = 2
NSUB = 16
NW = NCORES * NSUB
L = 16

NF = 10000
NP = 10240
RPW = NP // NW
D = 128
IPC = 128


def _gather_sum_sc(table, nbr_flat, K, KP, name):
    NT = table.shape[0]
    RC = IPC // KP
    nch = (RPW * KP) // IPC
    assert RPW % RC == 0 and nch % 2 == 0

    def body(table_ref, nbr_ref, out_ref, idx_v, buf0, buf1, acc_v, sem0, sem1):
        wid = lax.axis_index("s") * NCORES + lax.axis_index("c")
        base = wid * RPW

        pltpu.sync_copy(nbr_ref.at[pl.ds(base * KP, RPW * KP)], idx_v)
        lane = lax.iota(jnp.int32, L)

        def transform(i, _):
            v = idx_v[pl.ds(i * L, L)]
            p = lane + i * L
            t = jnp.minimum(p & (KP - 1), K - 1)
            idx_v[pl.ds(i * L, L)] = v * K + t
            return 0

        lax.fori_loop(0, (RPW * KP) // L, transform, 0)

        bufs = (buf0, buf1)
        sems = (sem0, sem1)

        def fire(c, b):
            return pltpu.async_copy(
                table_ref.at[idx_v.at[pl.ds(c * IPC, IPC)]], bufs[b], sems[b]
            )

        def wait(c, b):
            pltpu.make_async_copy(
                table_ref.at[idx_v.at[pl.ds(c * IPC, IPC)]], bufs[b], sems[b]
            ).wait()

        def consume(c, b):
            buf = bufs[b]

            def row_body(rr, _):
                accs = [None] * (D // L)
                for t in range(K):
                    row = rr * KP + t
                    for j in range(D // L):
                        v = buf[row, pl.ds(j * L, L)]
                        accs[j] = v if t == 0 else accs[j] + v
                for j in range(D // L):
                    acc_v[c * RC + rr, pl.ds(j * L, L)] = accs[j]
                return 0

            lax.fori_loop(0, RC, row_body, 0)

        fire(0, 0)
        fire(1, 1)

        def chunk_body(c2, _):
            for b in range(2):
                c = c2 * 2 + b
                wait(c, b)
                consume(c, b)
                fire(c + 2, b)
            return 0

        lax.fori_loop(0, nch // 2 - 1, chunk_body, 0)
        for b in range(2):
            c = nch - 2 + b
            wait(c, b)
            consume(c, b)

        pltpu.sync_copy(acc_v, out_ref.at[pl.ds(base, RPW)])

    return pl.kernel(
        body,
        out_type=jax.ShapeDtypeStruct((NP, D), jnp.float32),
        mesh=plsc.VectorSubcoreMesh(core_axis_name="c", subcore_axis_name="s"),
        scratch_types=[
            pltpu.VMEM((RPW * KP,), jnp.int32),
            pltpu.VMEM((IPC, D), jnp.float32),
            pltpu.VMEM((IPC, D), jnp.float32),
            pltpu.VMEM((RPW, D), jnp.float32),
            pltpu.SemaphoreType.DMA,
            pltpu.SemaphoreType.DMA,
        ],
        name=name,
    )(table, nbr_flat)


def _bn_stats(h, g, be):
    def body(h_ref, g_ref, be_ref, scale_ref, shift_ref):
        hv = h_ref[...]
        rows = lax.broadcasted_iota(jnp.int32, hv.shape, 0)
        hm = jnp.where(rows < NF, hv, 0.0)
        mu = jnp.sum(hm, axis=0, keepdims=True) * (1.0 / NF)
        ms = jnp.sum(hm * hm, axis=0, keepdims=True) * (1.0 / NF)
        var = ms - mu * mu
        scale = g_ref[...] * lax.rsqrt(var + 1e-5)
        scale_ref[...] = scale
        shift_ref[...] = be_ref[...] - mu * scale

    return pl.pallas_call(
        body,
        out_shape=(
            jax.ShapeDtypeStruct((1, D), jnp.float32),
            jax.ShapeDtypeStruct((1, D), jnp.float32),
        ),
    )(h, g.reshape(1, D), be.reshape(1, D))


def _matmul(x, w):
    def body(x_ref, w_ref, o_ref):
        o_ref[...] = jnp.dot(x_ref[...], w_ref[...],
                             preferred_element_type=jnp.float32)

    return pl.pallas_call(
        body,
        out_shape=jax.ShapeDtypeStruct((x.shape[0], w.shape[1]), jnp.float32),
    )(x, w)


def _bn_relu_matmul(h, scale, shift, w, h2=None, w2=None, br=1024):
    m = h.shape[0]
    n = w.shape[1]

    def body(h_ref, sc_ref, sh_ref, w_ref, *rest):
        if h2 is not None:
            h2_ref, w2_ref, o_ref = rest
        else:
            (o_ref,) = rest
        a = jnp.maximum(h_ref[...] * sc_ref[...] + sh_ref[...], 0.0)
        acc = jnp.dot(a, w_ref[...], preferred_element_type=jnp.float32)
        if h2 is not None:
            acc = acc + jnp.dot(h2_ref[...], w2_ref[...],
                                preferred_element_type=jnp.float32)
        o_ref[...] = acc

    in_specs = [
        pl.BlockSpec((br, D), lambda i: (i, 0)),
        pl.BlockSpec((1, D), lambda i: (0, 0)),
        pl.BlockSpec((1, D), lambda i: (0, 0)),
        pl.BlockSpec((D, n), lambda i: (0, 0)),
    ]
    args = [h, scale, shift, w]
    if h2 is not None:
        in_specs += [
            pl.BlockSpec((br, D), lambda i: (i, 0)),
            pl.BlockSpec((D, n), lambda i: (0, 0)),
        ]
        args += [h2, w2]

    return pl.pallas_call(
        body,
        grid=(m // br,),
        in_specs=in_specs,
        out_specs=pl.BlockSpec((br, n), lambda i: (i, 0)),
        out_shape=jax.ShapeDtypeStruct((m, n), jnp.float32),
    )(*args)


def _bn_relu(h, scale, shift):
    def body(h_ref, sc_ref, sh_ref, o_ref):
        o_ref[...] = jnp.maximum(h_ref[...] * sc_ref[...] + sh_ref[...], 0.0)

    return pl.pallas_call(
        body,
        out_shape=jax.ShapeDtypeStruct(h.shape, jnp.float32),
    )(h, scale, shift)


def _pack_nbr(nbr, KP):
    K = nbr.shape[0]
    nbr_t = jnp.pad(nbr, ((0, 0), (0, NP - nbr.shape[1]))).T
    if KP > K:
        nbr_t = jnp.pad(nbr_t, ((0, 0), (0, KP - K)))
    return nbr_t.reshape(NP * KP)


def kernel(x, x_skip, up_nbr, nbr1, nbr2, W_up, b_up, g_up, be_up,
           W1, g1, be1, W2, g2, be2):
    del b_up
    w_up_c = W_up.transpose(1, 0, 2).reshape(D, 8 * D)
    w1a = W1[:, :D, :].transpose(1, 0, 2).reshape(D, 27 * D)
    w1b = W1[:, D:, :].transpose(1, 0, 2).reshape(D, 27 * D)
    w2c = W2.transpose(1, 0, 2).reshape(D, 27 * D)

    xp = jnp.pad(x, ((0, 1280 - x.shape[0]), (0, 0)))
    xs_p = jnp.pad(x_skip, ((0, NP - NF), (0, 0)))
    up_f = _pack_nbr(up_nbr, 8)
    nbr1_f = _pack_nbr(nbr1, 32)
    nbr2_f = _pack_nbr(nbr2, 32)

    yup = _matmul(xp, w_up_c).reshape(1280 * 8, D)
    h1 = _gather_sum_sc(yup, up_f, 8, 8, "sc_up_gather")
    sc1, sh1 = _bn_stats(h1, g_up, be_up)

    y1 = _bn_relu_matmul(h1, sc1, sh1, w1a, h2=xs_p, w2=w1b).reshape(NP * 27, D)
    h2 = _gather_sum_sc(y1, nbr1_f, 27, 32, "sc_conv1_gather")
    sc2, sh2 = _bn_stats(h2, g1, be1)

    y2 = _bn_relu_matmul(h2, sc2, sh2, w2c).reshape(NP * 27, D)
    h3 = _gather_sum_sc(y2, nbr2_f, 27, 32, "sc_conv2_gather")
    sc3, sh3 = _bn_stats(h3, g2, be2)

    out = _bn_relu(h3, sc3, sh3)
    return out[:NF]


# speedup vs baseline: 2.4858x; 2.4858x over previous
import functools

import jax
import jax.numpy as jnp
from jax import lax
from jax.experimental import pallas as pl
from jax.experimental.pallas import tpu as pltpu
from jax.experimental.pallas import tpu_sc as plsc

NCORES = 2
NSUB = 16
NW = NCORES * NSUB
L = 16

NF = 10000
NP = 10240
RPW = NP // NW
D = 128
IPC = 128


def _gather_sum_sc(table, nbr_flat, K, KP, name):
    NT = table.shape[0]
    RC = IPC // KP
    nch = (RPW * KP) // IPC
    assert RPW % RC == 0 and nch % 2 == 0

    def body(table_ref, nbr_ref, out_ref, idx_v, buf0, buf1, acc_v, sem0, sem1):
        wid = lax.axis_index("s") * NCORES + lax.axis_index("c")
        base = wid * RPW

        pltpu.sync_copy(nbr_ref.at[pl.ds(base * KP, RPW * KP)], idx_v)
        lane = lax.iota(jnp.int32, L)

        def transform(i, _):
            v = idx_v[pl.ds(i * L, L)]
            p = lane + i * L
            t = jnp.minimum(p & (KP - 1), K - 1)
            idx_v[pl.ds(i * L, L)] = v * K + t
            return 0

        lax.fori_loop(0, (RPW * KP) // L, transform, 0)

        bufs = (buf0, buf1)
        sems = (sem0, sem1)

        def fire(c, b):
            return pltpu.async_copy(
                table_ref.at[idx_v.at[pl.ds(c * IPC, IPC)]], bufs[b], sems[b]
            )

        def wait(c, b):
            pltpu.make_async_copy(
                table_ref.at[idx_v.at[pl.ds(c * IPC, IPC)]], bufs[b], sems[b]
            ).wait()

        def consume(c, b):
            buf = bufs[b]

            def row_body(rr, _):
                accs = [None] * (D // L)
                for t in range(K):
                    row = rr * KP + t
                    for j in range(D // L):
                        v = buf[row, pl.ds(j * L, L)]
                        accs[j] = v if t == 0 else accs[j] + v
                for j in range(D // L):
                    acc_v[c * RC + rr, pl.ds(j * L, L)] = accs[j]
                return 0

            lax.fori_loop(0, RC, row_body, 0)

        fire(0, 0)
        fire(1, 1)

        def chunk_body(c2, _):
            for b in range(2):
                c = c2 * 2 + b
                wait(c, b)
                consume(c, b)
                fire(c + 2, b)
            return 0

        lax.fori_loop(0, nch // 2 - 1, chunk_body, 0)
        for b in range(2):
            c = nch - 2 + b
            wait(c, b)
            consume(c, b)

        pltpu.sync_copy(acc_v, out_ref.at[pl.ds(base, RPW)])

    return pl.kernel(
        body,
        out_type=jax.ShapeDtypeStruct((NP, D), jnp.float32),
        mesh=plsc.VectorSubcoreMesh(core_axis_name="c", subcore_axis_name="s"),
        scratch_types=[
            pltpu.VMEM((RPW * KP,), jnp.int32),
            pltpu.VMEM((IPC, D), jnp.float32),
            pltpu.VMEM((IPC, D), jnp.float32),
            pltpu.VMEM((RPW, D), jnp.float32),
            pltpu.SemaphoreType.DMA,
            pltpu.SemaphoreType.DMA,
        ],
        name=name,
    )(table, nbr_flat)


def _gather_rows_sc(table, nbr_w, K, W, name):
    IPC_G = 80
    CPT = RPW // IPC_G
    nch = K * CPT
    IWP = nbr_w.shape[2]
    assert nch % 2 == 0 and IWP >= K * RPW

    def body(table_ref, nbr_ref, out_ref, idx_v, buf0, buf1, sem0, sem1):
        wid = lax.axis_index("s") * NCORES + lax.axis_index("c")
        base = wid * RPW

        pltpu.sync_copy(nbr_ref.at[wid, 0], idx_v)

        bufs = (buf0, buf1)
        sems = (sem0, sem1)

        def fire(c, b):
            return pltpu.async_copy(
                table_ref.at[idx_v.at[pl.ds(c * IPC_G, IPC_G)]],
                bufs[b], sems[b])

        def wait(c, b):
            pltpu.make_async_copy(
                table_ref.at[idx_v.at[pl.ds(c * IPC_G, IPC_G)]],
                bufs[b], sems[b]).wait()

        def drain(c, b):
            t = c // CPT
            cc = c - t * CPT
            pltpu.sync_copy(bufs[b],
                            out_ref.at[t, pl.ds(base + cc * IPC_G, IPC_G)])

        fire(0, 0)
        fire(1, 1)

        def chunk_body(c2, _):
            for b in range(2):
                c = c2 * 2 + b
                wait(c, b)
                drain(c, b)
                fire(c + 2, b)
            return 0

        lax.fori_loop(0, nch // 2 - 1, chunk_body, 0)
        for b in range(2):
            c = nch - 2 + b
            wait(c, b)
            drain(c, b)

    return pl.kernel(
        body,
        out_type=jax.ShapeDtypeStruct((K, NP, W), jnp.float32),
        mesh=plsc.VectorSubcoreMesh(core_axis_name="c", subcore_axis_name="s"),
        scratch_types=[
            pltpu.VMEM((IWP,), jnp.int32),
            pltpu.VMEM((IPC_G, W), jnp.float32),
            pltpu.VMEM((IPC_G, W), jnp.float32),
            pltpu.SemaphoreType.DMA,
            pltpu.SemaphoreType.DMA,
        ],
        name=name,
    )(table, nbr_w)


def _tap_matmul(g3, w3, br=512):
    T, _, W = g3.shape

    def body(g_ref, w_ref, o_ref):
        t = pl.program_id(1)

        @pl.when(t == 0)
        def _():
            o_ref[...] = jnp.zeros_like(o_ref)

        o_ref[...] += jnp.dot(g_ref[0], w_ref[0],
                              preferred_element_type=jnp.float32)

    return pl.pallas_call(
        body,
        grid=(NP // br, T),
        in_specs=[
            pl.BlockSpec((1, br, W), lambda i, t: (t, i, 0)),
            pl.BlockSpec((1, W, D), lambda i, t: (t, 0, 0)),
        ],
        out_specs=pl.BlockSpec((br, D), lambda i, t: (i, 0)),
        out_shape=jax.ShapeDtypeStruct((NP, D), jnp.float32),
    )(g3, w3)


def _bn_relu_cat(h, scale, shift, xs, br=1024):
    def body(h_ref, sc_ref, sh_ref, xs_ref, o_ref):
        o_ref[:, 0:D] = jnp.maximum(h_ref[...] * sc_ref[...] + sh_ref[...], 0.0)
        o_ref[:, D:2 * D] = xs_ref[...]

    return pl.pallas_call(
        body,
        grid=(NP // br,),
        in_specs=[
            pl.BlockSpec((br, D), lambda i: (i, 0)),
            pl.BlockSpec((1, D), lambda i: (0, 0)),
            pl.BlockSpec((1, D), lambda i: (0, 0)),
            pl.BlockSpec((br, D), lambda i: (i, 0)),
        ],
        out_specs=pl.BlockSpec((br, 2 * D), lambda i: (i, 0)),
        out_shape=jax.ShapeDtypeStruct((NP, 2 * D), jnp.float32),
    )(h, scale, shift, xs)


def _bn_stats(h, g, be):
    def body(h_ref, g_ref, be_ref, scale_ref, shift_ref):
        hv = h_ref[...]
        rows = lax.broadcasted_iota(jnp.int32, hv.shape, 0)
        hm = jnp.where(rows < NF, hv, 0.0)
        mu = jnp.sum(hm, axis=0, keepdims=True) * (1.0 / NF)
        ms = jnp.sum(hm * hm, axis=0, keepdims=True) * (1.0 / NF)
        var = ms - mu * mu
        scale = g_ref[...] * lax.rsqrt(var + 1e-5)
        scale_ref[...] = scale
        shift_ref[...] = be_ref[...] - mu * scale

    return pl.pallas_call(
        body,
        out_shape=(
            jax.ShapeDtypeStruct((1, D), jnp.float32),
            jax.ShapeDtypeStruct((1, D), jnp.float32),
        ),
    )(h, g.reshape(1, D), be.reshape(1, D))


def _matmul(x, w):
    def body(x_ref, w_ref, o_ref):
        o_ref[...] = jnp.dot(x_ref[...], w_ref[...],
                             preferred_element_type=jnp.float32)

    return pl.pallas_call(
        body,
        out_shape=jax.ShapeDtypeStruct((x.shape[0], w.shape[1]), jnp.float32),
    )(x, w)


def _bn_relu(h, scale, shift):
    def body(h_ref, sc_ref, sh_ref, o_ref):
        o_ref[...] = jnp.maximum(h_ref[...] * sc_ref[...] + sh_ref[...], 0.0)

    return pl.pallas_call(
        body,
        out_shape=jax.ShapeDtypeStruct(h.shape, jnp.float32),
    )(h, scale, shift)


def _pack_nbr(nbr, KP):
    K = nbr.shape[0]
    nbr_t = jnp.pad(nbr, ((0, 0), (0, NP - nbr.shape[1]))).T
    if KP > K:
        nbr_t = jnp.pad(nbr_t, ((0, 0), (0, KP - K)))
    return nbr_t.reshape(NP * KP)


def _pack_nbr_worker(nbr):
    K = nbr.shape[0]
    iw = K * RPW
    iwp = (iw + 127) // 128 * 128
    t = jnp.pad(nbr, ((0, 0), (0, NP - nbr.shape[1])))
    t = t.reshape(K, NW, RPW).transpose(1, 0, 2).reshape(NW, iw)
    return jnp.pad(t, ((0, 0), (0, iwp - iw))).reshape(NW, 1, iwp)


def kernel(x, x_skip, up_nbr, nbr1, nbr2, W_up, b_up, g_up, be_up,
           W1, g1, be1, W2, g2, be2):
    del b_up
    w_up_c = W_up.transpose(1, 0, 2).reshape(D, 8 * D)

    xp = jnp.pad(x, ((0, 1280 - x.shape[0]), (0, 0)))
    xs_p = jnp.pad(x_skip, ((0, NP - NF), (0, 0)))
    up_f = _pack_nbr(up_nbr, 8)
    nbr1_p = _pack_nbr_worker(nbr1)
    nbr2_p = _pack_nbr_worker(nbr2)

    yup = _matmul(xp, w_up_c).reshape(1280 * 8, D)
    h1 = _gather_sum_sc(yup, up_f, 8, 8, "sc_up_gather")
    sc1, sh1 = _bn_stats(h1, g_up, be_up)

    a1 = _bn_relu_cat(h1, sc1, sh1, xs_p)
    gt1 = _gather_rows_sc(a1, nbr1_p, 27, 2 * D, "sc_conv1_gather")
    h2 = _tap_matmul(gt1, W1)
    sc2, sh2 = _bn_stats(h2, g1, be1)

    a2 = _bn_relu(h2, sc2, sh2)
    gt2 = _gather_rows_sc(a2, nbr2_p, 27, D, "sc_conv2_gather")
    h3 = _tap_matmul(gt2, W2)
    sc3, sh3 = _bn_stats(h3, g2, be2)

    out = _bn_relu(h3, sc3, sh3)
    return out[:NF]


# speedup vs baseline: 2.5343x; 1.0195x over previous
import functools

import jax
import jax.numpy as jnp
from jax import lax
from jax.experimental import pallas as pl
from jax.experimental.pallas import tpu as pltpu
from jax.experimental.pallas import tpu_sc as plsc

NCORES = 2
NSUB = 16
NW = NCORES * NSUB
L = 16

NF = 10000
NP = 10240
RPW = NP // NW
D = 128
IPC = 128


def _gather_sum_sc(table, nbr_flat, K, KP, name):
    NT = table.shape[0]
    RC = IPC // KP
    nch = (RPW * KP) // IPC
    assert RPW % RC == 0 and nch % 2 == 0

    def body(table_ref, nbr_ref, out_ref, idx_v, buf0, buf1, acc_v, sem0, sem1):
        wid = lax.axis_index("s") * NCORES + lax.axis_index("c")
        base = wid * RPW

        pltpu.sync_copy(nbr_ref.at[pl.ds(base * KP, RPW * KP)], idx_v)
        lane = lax.iota(jnp.int32, L)

        def transform(i, _):
            v = idx_v[pl.ds(i * L, L)]
            p = lane + i * L
            t = jnp.minimum(p & (KP - 1), K - 1)
            idx_v[pl.ds(i * L, L)] = v * K + t
            return 0

        lax.fori_loop(0, (RPW * KP) // L, transform, 0)

        bufs = (buf0, buf1)
        sems = (sem0, sem1)

        def fire(c, b):
            return pltpu.async_copy(
                table_ref.at[idx_v.at[pl.ds(c * IPC, IPC)]], bufs[b], sems[b]
            )

        def wait(c, b):
            pltpu.make_async_copy(
                table_ref.at[idx_v.at[pl.ds(c * IPC, IPC)]], bufs[b], sems[b]
            ).wait()

        def consume(c, b):
            buf = bufs[b]

            def row_body(rr, _):
                accs = [None] * (D // L)
                for t in range(K):
                    row = rr * KP + t
                    for j in range(D // L):
                        v = buf[row, pl.ds(j * L, L)]
                        accs[j] = v if t == 0 else accs[j] + v
                for j in range(D // L):
                    acc_v[c * RC + rr, pl.ds(j * L, L)] = accs[j]
                return 0

            lax.fori_loop(0, RC, row_body, 0)

        fire(0, 0)
        fire(1, 1)

        def chunk_body(c2, _):
            for b in range(2):
                c = c2 * 2 + b
                wait(c, b)
                consume(c, b)
                fire(c + 2, b)
            return 0

        lax.fori_loop(0, nch // 2 - 1, chunk_body, 0)
        for b in range(2):
            c = nch - 2 + b
            wait(c, b)
            consume(c, b)

        pltpu.sync_copy(acc_v, out_ref.at[pl.ds(base, RPW)])

    return pl.kernel(
        body,
        out_type=jax.ShapeDtypeStruct((NP, D), jnp.float32),
        mesh=plsc.VectorSubcoreMesh(core_axis_name="c", subcore_axis_name="s"),
        scratch_types=[
            pltpu.VMEM((RPW * KP,), jnp.int32),
            pltpu.VMEM((IPC, D), jnp.float32),
            pltpu.VMEM((IPC, D), jnp.float32),
            pltpu.VMEM((RPW, D), jnp.float32),
            pltpu.SemaphoreType.DMA,
            pltpu.SemaphoreType.DMA,
        ],
        name=name,
    )(table, nbr_flat)


def _gather_rows_sc(table, nbr_w, K, W, name):
    IPC_G = 80
    CPT = RPW // IPC_G
    nch = K * CPT
    NS_B = 4
    IWP = nbr_w.shape[2]
    assert nch % NS_B == 0 and IWP >= K * RPW
    dt = table.dtype

    def body(table_ref, nbr_ref, out_ref, idx_v,
             buf0, buf1, buf2, buf3, gs0, gs1, gs2, gs3, ws0, ws1, ws2, ws3):
        wid = lax.axis_index("s") * NCORES + lax.axis_index("c")
        base = wid * RPW

        pltpu.sync_copy(nbr_ref.at[wid, 0], idx_v)

        bufs = (buf0, buf1, buf2, buf3)
        gsems = (gs0, gs1, gs2, gs3)
        wsems = (ws0, ws1, ws2, ws3)

        def g_desc(c, b):
            return pltpu.make_async_copy(
                table_ref.at[idx_v.at[pl.ds(c * IPC_G, IPC_G)]],
                bufs[b], gsems[b])

        def w_desc(c, b):
            t = c // CPT
            cc = c - t * CPT
            return pltpu.make_async_copy(
                bufs[b], out_ref.at[t, pl.ds(base + cc * IPC_G, IPC_G)],
                wsems[b])

        for b in range(NS_B):
            g_desc(b, b).start()

        def round_body(r, _):
            c0 = r * NS_B
            for b in range(NS_B):
                g_desc(c0 + b, b).wait()
                w_desc(c0 + b, b).start()
            for b in range(NS_B):
                w_desc(c0 + b, b).wait()
                g_desc(c0 + NS_B + b, b).start()
            return 0

        lax.fori_loop(0, nch // NS_B - 1, round_body, 0)
        c0 = nch - NS_B
        for b in range(NS_B):
            g_desc(c0 + b, b).wait()
            w_desc(c0 + b, b).start()
        for b in range(NS_B):
            w_desc(c0 + b, b).wait()

    return pl.kernel(
        body,
        out_type=jax.ShapeDtypeStruct((K, NP, W), dt),
        mesh=plsc.VectorSubcoreMesh(core_axis_name="c", subcore_axis_name="s"),
        scratch_types=[
            pltpu.VMEM((IWP,), jnp.int32),
        ] + [pltpu.VMEM((IPC_G, W), dt) for _ in range(NS_B)]
          + [pltpu.SemaphoreType.DMA] * (2 * NS_B),
        name=name,
    )(table, nbr_w)


def _tap_matmul(g3, w3, br=512):
    T, _, W = g3.shape

    def body(g_ref, w_ref, o_ref):
        t = pl.program_id(1)

        @pl.when(t == 0)
        def _():
            o_ref[...] = jnp.zeros_like(o_ref)

        o_ref[...] += jnp.dot(g_ref[0], w_ref[0],
                              preferred_element_type=jnp.float32)

    return pl.pallas_call(
        body,
        grid=(NP // br, T),
        in_specs=[
            pl.BlockSpec((1, br, W), lambda i, t: (t, i, 0)),
            pl.BlockSpec((1, W, D), lambda i, t: (t, 0, 0)),
        ],
        out_specs=pl.BlockSpec((br, D), lambda i, t: (i, 0)),
        out_shape=jax.ShapeDtypeStruct((NP, D), jnp.float32),
    )(g3, w3)


def _bn_relu_cat(h, scale, shift, xs, br=1024):
    def body(h_ref, sc_ref, sh_ref, xs_ref, o_ref):
        o_ref[:, 0:D] = jnp.maximum(h_ref[...] * sc_ref[...] + sh_ref[...], 0.0)
        o_ref[:, D:2 * D] = xs_ref[...]

    return pl.pallas_call(
        body,
        grid=(NP // br,),
        in_specs=[
            pl.BlockSpec((br, D), lambda i: (i, 0)),
            pl.BlockSpec((1, D), lambda i: (0, 0)),
            pl.BlockSpec((1, D), lambda i: (0, 0)),
            pl.BlockSpec((br, D), lambda i: (i, 0)),
        ],
        out_specs=pl.BlockSpec((br, 2 * D), lambda i: (i, 0)),
        out_shape=jax.ShapeDtypeStruct((NP, 2 * D), jnp.float32),
    )(h, scale, shift, xs)


def _bn_stats(h, g, be):
    def body(h_ref, g_ref, be_ref, scale_ref, shift_ref):
        hv = h_ref[...]
        rows = lax.broadcasted_iota(jnp.int32, hv.shape, 0)
        hm = jnp.where(rows < NF, hv, 0.0)
        mu = jnp.sum(hm, axis=0, keepdims=True) * (1.0 / NF)
        ms = jnp.sum(hm * hm, axis=0, keepdims=True) * (1.0 / NF)
        var = ms - mu * mu
        scale = g_ref[...] * lax.rsqrt(var + 1e-5)
        scale_ref[...] = scale
        shift_ref[...] = be_ref[...] - mu * scale

    return pl.pallas_call(
        body,
        out_shape=(
            jax.ShapeDtypeStruct((1, D), jnp.float32),
            jax.ShapeDtypeStruct((1, D), jnp.float32),
        ),
    )(h, g.reshape(1, D), be.reshape(1, D))


def _matmul(x, w):
    def body(x_ref, w_ref, o_ref):
        o_ref[...] = jnp.dot(x_ref[...], w_ref[...],
                             preferred_element_type=jnp.float32)

    return pl.pallas_call(
        body,
        out_shape=jax.ShapeDtypeStruct((x.shape[0], w.shape[1]), jnp.float32),
    )(x, w)


def _bn_relu(h, scale, shift):
    def body(h_ref, sc_ref, sh_ref, o_ref):
        o_ref[...] = jnp.maximum(h_ref[...] * sc_ref[...] + sh_ref[...], 0.0)

    return pl.pallas_call(
        body,
        out_shape=jax.ShapeDtypeStruct(h.shape, jnp.float32),
    )(h, scale, shift)


def _pack_nbr(nbr, KP):
    K = nbr.shape[0]
    nbr_t = jnp.pad(nbr, ((0, 0), (0, NP - nbr.shape[1]))).T
    if KP > K:
        nbr_t = jnp.pad(nbr_t, ((0, 0), (0, KP - K)))
    return nbr_t.reshape(NP * KP)


def _pack_nbr_worker(nbr):
    K = nbr.shape[0]
    iw = K * RPW
    iwp = (iw + 127) // 128 * 128
    t = jnp.pad(nbr, ((0, 0), (0, NP - nbr.shape[1])))
    t = t.reshape(K, NW, RPW).transpose(1, 0, 2).reshape(NW, iw)
    return jnp.pad(t, ((0, 0), (0, iwp - iw))).reshape(NW, 1, iwp)


def kernel(x, x_skip, up_nbr, nbr1, nbr2, W_up, b_up, g_up, be_up,
           W1, g1, be1, W2, g2, be2):
    del b_up
    w_up_c = W_up.transpose(1, 0, 2).reshape(D, 8 * D)

    xp = jnp.pad(x, ((0, 1280 - x.shape[0]), (0, 0)))
    xs_p = jnp.pad(x_skip, ((0, NP - NF), (0, 0)))
    up_f = _pack_nbr(up_nbr, 8)
    nbr1_p = _pack_nbr_worker(nbr1)
    nbr2_p = _pack_nbr_worker(nbr2)

    yup = _matmul(xp, w_up_c).reshape(1280 * 8, D)
    h1 = _gather_sum_sc(yup, up_f, 8, 8, "sc_up_gather")
    sc1, sh1 = _bn_stats(h1, g_up, be_up)

    a1 = _bn_relu_cat(h1, sc1, sh1, xs_p)
    gt1 = _gather_rows_sc(a1, nbr1_p, 27, 2 * D, "sc_conv1_gather")
    h2 = _tap_matmul(gt1, W1)
    sc2, sh2 = _bn_stats(h2, g1, be1)

    a2 = _bn_relu(h2, sc2, sh2)
    gt2 = _gather_rows_sc(a2, nbr2_p, 27, D, "sc_conv2_gather")
    h3 = _tap_matmul(gt2, W2)
    sc3, sh3 = _bn_stats(h3, g2, be2)

    out = _bn_relu(h3, sc3, sh3)
    return out[:NF]


# speedup vs baseline: 3.9113x; 1.5433x over previous
import functools

import jax
import jax.numpy as jnp
from jax import lax
from jax.experimental import pallas as pl
from jax.experimental.pallas import tpu as pltpu
from jax.experimental.pallas import tpu_sc as plsc

NCORES = 2
NSUB = 16
NW = NCORES * NSUB
L = 16

NF = 10000
NP = 10240
RPW = NP // NW
D = 128
IPC = 128


def _gather_sum_sc(table, nbr_flat, K, KP, name):
    NT = table.shape[0]
    RC = IPC // KP
    nch = (RPW * KP) // IPC
    assert RPW % RC == 0 and nch % 2 == 0

    def body(table_ref, nbr_ref, out_ref, idx_v, buf0, buf1, acc_v, sem0, sem1):
        wid = lax.axis_index("s") * NCORES + lax.axis_index("c")
        base = wid * RPW

        pltpu.sync_copy(nbr_ref.at[pl.ds(base * KP, RPW * KP)], idx_v)
        lane = lax.iota(jnp.int32, L)

        def transform(i, _):
            v = idx_v[pl.ds(i * L, L)]
            p = lane + i * L
            t = jnp.minimum(p & (KP - 1), K - 1)
            idx_v[pl.ds(i * L, L)] = v * K + t
            return 0

        lax.fori_loop(0, (RPW * KP) // L, transform, 0)

        bufs = (buf0, buf1)
        sems = (sem0, sem1)

        def fire(c, b):
            return pltpu.async_copy(
                table_ref.at[idx_v.at[pl.ds(c * IPC, IPC)]], bufs[b], sems[b]
            )

        def wait(c, b):
            pltpu.make_async_copy(
                table_ref.at[idx_v.at[pl.ds(c * IPC, IPC)]], bufs[b], sems[b]
            ).wait()

        def consume(c, b):
            buf = bufs[b]

            def row_body(rr, _):
                accs = [None] * (D // L)
                for t in range(K):
                    row = rr * KP + t
                    for j in range(D // L):
                        v = buf[row, pl.ds(j * L, L)]
                        accs[j] = v if t == 0 else accs[j] + v
                for j in range(D // L):
                    acc_v[c * RC + rr, pl.ds(j * L, L)] = accs[j]
                return 0

            lax.fori_loop(0, RC, row_body, 0)

        fire(0, 0)
        fire(1, 1)

        def chunk_body(c2, _):
            for b in range(2):
                c = c2 * 2 + b
                wait(c, b)
                consume(c, b)
                fire(c + 2, b)
            return 0

        lax.fori_loop(0, nch // 2 - 1, chunk_body, 0)
        for b in range(2):
            c = nch - 2 + b
            wait(c, b)
            consume(c, b)

        pltpu.sync_copy(acc_v, out_ref.at[pl.ds(base, RPW)])

    return pl.kernel(
        body,
        out_type=jax.ShapeDtypeStruct((NP, D), jnp.float32),
        mesh=plsc.VectorSubcoreMesh(core_axis_name="c", subcore_axis_name="s"),
        scratch_types=[
            pltpu.VMEM((RPW * KP,), jnp.int32),
            pltpu.VMEM((IPC, D), jnp.float32),
            pltpu.VMEM((IPC, D), jnp.float32),
            pltpu.VMEM((RPW, D), jnp.float32),
            pltpu.SemaphoreType.DMA,
            pltpu.SemaphoreType.DMA,
        ],
        name=name,
    )(table, nbr_flat)


def _gather_rows_sc(table, nbr_w, K, W, name):
    IPC_G = 80
    CPT = RPW // IPC_G
    nch = K * CPT
    NS_B = 4
    IWP = nbr_w.shape[2]
    assert nch % NS_B == 0 and IWP >= K * RPW
    dt = table.dtype

    def body(table_ref, nbr_ref, out_ref, idx_v,
             buf0, buf1, buf2, buf3, gs0, gs1, gs2, gs3, ws0, ws1, ws2, ws3):
        wid = lax.axis_index("s") * NCORES + lax.axis_index("c")
        base = wid * RPW

        pltpu.sync_copy(nbr_ref.at[wid, 0], idx_v)

        bufs = (buf0, buf1, buf2, buf3)
        gsems = (gs0, gs1, gs2, gs3)
        wsems = (ws0, ws1, ws2, ws3)

        def g_desc(c, b):
            return pltpu.make_async_copy(
                table_ref.at[idx_v.at[pl.ds(c * IPC_G, IPC_G)]],
                bufs[b], gsems[b])

        def w_desc(c, b):
            t = c // CPT
            cc = c - t * CPT
            return pltpu.make_async_copy(
                bufs[b], out_ref.at[t, pl.ds(base + cc * IPC_G, IPC_G)],
                wsems[b])

        for b in range(NS_B):
            g_desc(b, b).start()

        def round_body(r, _):
            c0 = r * NS_B
            for b in range(NS_B):
                g_desc(c0 + b, b).wait()
                w_desc(c0 + b, b).start()
            for b in range(NS_B):
                w_desc(c0 + b, b).wait()
                g_desc(c0 + NS_B + b, b).start()
            return 0

        lax.fori_loop(0, nch // NS_B - 1, round_body, 0)
        c0 = nch - NS_B
        for b in range(NS_B):
            g_desc(c0 + b, b).wait()
            w_desc(c0 + b, b).start()
        for b in range(NS_B):
            w_desc(c0 + b, b).wait()

    return pl.kernel(
        body,
        out_type=jax.ShapeDtypeStruct((K, NP, W), dt),
        mesh=plsc.VectorSubcoreMesh(core_axis_name="c", subcore_axis_name="s"),
        scratch_types=[
            pltpu.VMEM((IWP,), jnp.int32),
        ] + [pltpu.VMEM((IPC_G, W), dt) for _ in range(NS_B)]
          + [pltpu.SemaphoreType.DMA] * (2 * NS_B),
        name=name,
    )(table, nbr_w)


def _pack16(lo, hi):
    ulo = lax.bitcast_convert_type(
        lo.astype(jnp.bfloat16).astype(jnp.float32), jnp.uint32)
    uhi = lax.bitcast_convert_type(
        hi.astype(jnp.bfloat16).astype(jnp.float32), jnp.uint32)
    return lax.bitcast_convert_type(uhi | (ulo >> 16), jnp.float32)


def _unpack16(w):
    u = lax.bitcast_convert_type(w, jnp.uint32)
    hi = lax.bitcast_convert_type(u & jnp.uint32(0xFFFF0000), jnp.float32)
    lo = lax.bitcast_convert_type(u << 16, jnp.float32)
    return lo, hi


def _tap_matmul(g3, w3, packed, br=512):
    T, _, Wp = g3.shape

    def body(g_ref, w_ref, o_ref):
        acc = jnp.zeros((br, D), jnp.float32)
        for t in range(T):
            wv = w_ref[t]
            if packed:
                lo, hi = _unpack16(g_ref[t])
                acc += jnp.dot(lo.astype(jnp.bfloat16), wv[0:Wp],
                               preferred_element_type=jnp.float32)
                acc += jnp.dot(hi.astype(jnp.bfloat16), wv[Wp:2 * Wp],
                               preferred_element_type=jnp.float32)
            else:
                acc += jnp.dot(g_ref[t], wv,
                               preferred_element_type=jnp.float32)
        o_ref[...] = acc

    kw = 2 * Wp if packed else Wp
    return pl.pallas_call(
        body,
        grid=(NP // br,),
        in_specs=[
            pl.BlockSpec((T, br, Wp), lambda i: (0, i, 0)),
            pl.BlockSpec((T, kw, D), lambda i: (0, 0, 0)),
        ],
        out_specs=pl.BlockSpec((br, D), lambda i: (i, 0)),
        out_shape=jax.ShapeDtypeStruct((NP, D), jnp.float32),
    )(g3, w3)


def _bn_relu_cat(h, scale, shift, xs, br=1024):
    def body(h_ref, sc_ref, sh_ref, xs_ref, o_ref):
        a = jnp.maximum(h_ref[...] * sc_ref[...] + sh_ref[...], 0.0)
        o_ref[...] = _pack16(a, xs_ref[...])

    return pl.pallas_call(
        body,
        grid=(NP // br,),
        in_specs=[
            pl.BlockSpec((br, D), lambda i: (i, 0)),
            pl.BlockSpec((1, D), lambda i: (0, 0)),
            pl.BlockSpec((1, D), lambda i: (0, 0)),
            pl.BlockSpec((br, D), lambda i: (i, 0)),
        ],
        out_specs=pl.BlockSpec((br, D), lambda i: (i, 0)),
        out_shape=jax.ShapeDtypeStruct((NP, D), jnp.float32),
    )(h, scale, shift, xs)


def _bn_stats(h, g, be):
    def body(h_ref, g_ref, be_ref, scale_ref, shift_ref):
        hv = h_ref[...]
        rows = lax.broadcasted_iota(jnp.int32, hv.shape, 0)
        hm = jnp.where(rows < NF, hv, 0.0)
        mu = jnp.sum(hm, axis=0, keepdims=True) * (1.0 / NF)
        ms = jnp.sum(hm * hm, axis=0, keepdims=True) * (1.0 / NF)
        var = ms - mu * mu
        scale = g_ref[...] * lax.rsqrt(var + 1e-5)
        scale_ref[...] = scale
        shift_ref[...] = be_ref[...] - mu * scale

    return pl.pallas_call(
        body,
        out_shape=(
            jax.ShapeDtypeStruct((1, D), jnp.float32),
            jax.ShapeDtypeStruct((1, D), jnp.float32),
        ),
    )(h, g.reshape(1, D), be.reshape(1, D))


def _matmul(x, w):
    def body(x_ref, w_ref, o_ref):
        o_ref[...] = jnp.dot(x_ref[...], w_ref[...],
                             preferred_element_type=jnp.float32)

    return pl.pallas_call(
        body,
        out_shape=jax.ShapeDtypeStruct((x.shape[0], w.shape[1]), jnp.float32),
    )(x, w)


def _bn_relu(h, scale, shift):
    def body(h_ref, sc_ref, sh_ref, o_ref):
        o_ref[...] = jnp.maximum(h_ref[...] * sc_ref[...] + sh_ref[...], 0.0)

    return pl.pallas_call(
        body,
        out_shape=jax.ShapeDtypeStruct(h.shape, jnp.float32),
    )(h, scale, shift)


def _pack_nbr(nbr, KP):
    K = nbr.shape[0]
    nbr_t = jnp.pad(nbr, ((0, 0), (0, NP - nbr.shape[1]))).T
    if KP > K:
        nbr_t = jnp.pad(nbr_t, ((0, 0), (0, KP - K)))
    return nbr_t.reshape(NP * KP)


def _pack_nbr_worker(nbr):
    K = nbr.shape[0]
    iw = K * RPW
    iwp = (iw + 127) // 128 * 128
    t = jnp.pad(nbr, ((0, 0), (0, NP - nbr.shape[1])))
    t = t.reshape(K, NW, RPW).transpose(1, 0, 2).reshape(NW, iw)
    return jnp.pad(t, ((0, 0), (0, iwp - iw))).reshape(NW, 1, iwp)


def kernel(x, x_skip, up_nbr, nbr1, nbr2, W_up, b_up, g_up, be_up,
           W1, g1, be1, W2, g2, be2):
    del b_up
    w_up_c = W_up.transpose(1, 0, 2).reshape(D, 8 * D)

    xp = jnp.pad(x, ((0, 1280 - x.shape[0]), (0, 0)))
    xs_p = jnp.pad(x_skip, ((0, NP - NF), (0, 0)))
    up_f = _pack_nbr(up_nbr, 8)
    nbr1_p = _pack_nbr_worker(nbr1)
    nbr2_p = _pack_nbr_worker(nbr2)

    yup = _matmul(xp, w_up_c).reshape(1280 * 8, D)
    h1 = _gather_sum_sc(yup, up_f, 8, 8, "sc_up_gather")
    sc1, sh1 = _bn_stats(h1, g_up, be_up)

    a1 = _bn_relu_cat(h1, sc1, sh1, xs_p)
    gt1 = _gather_rows_sc(a1, nbr1_p, 27, D, "sc_conv1_gather")
    h2 = _tap_matmul(gt1, W1.astype(jnp.bfloat16), packed=True)
    sc2, sh2 = _bn_stats(h2, g1, be1)

    a2 = _bn_relu(h2, sc2, sh2)
    gt2 = _gather_rows_sc(a2, nbr2_p, 27, D, "sc_conv2_gather")
    h3 = _tap_matmul(gt2, W2, packed=False)
    sc3, sh3 = _bn_stats(h3, g2, be2)

    out = _bn_relu(h3, sc3, sh3)
    return out[:NF]
